# Initial kernel scaffold; baseline (speedup 1.0000x reference)
import functools

import jax
import jax.numpy as jnp
from jax import lax
from jax.experimental import pallas as pl
from jax.experimental.pallas import tpu as pltpu
from jax.experimental.pallas import tpu_sc as plsc

N = 10000
E = 320000
D = 128
NC = 2
NS = 16
NW = NC * NS
CH = 128
EPW = E // NW
NCHUNK = -(-EPW // CH)
NPAD = 10240
RPT = NPAD // NS
ZC = RPT // CH
RB = 400
GRID = N // RB

_mesh = plsc.VectorSubcoreMesh(core_axis_name="c", subcore_axis_name="s",
                               num_cores=NC, num_subcores=NS)


def _deg_body(dst_hbm, out_hbm, idx_v, ones_v, zero_v, acc_sh, sem):
    c = lax.axis_index("c")
    s = lax.axis_index("s")
    wid = c * NS + s

    def _fill(i, _):
        ones_v[i, :] = jnp.ones((16,), jnp.float32)
        zero_v[i, :] = jnp.zeros((16,), jnp.float32)
        return 0

    lax.fori_loop(0, CH, _fill, 0)
    for k in range(ZC):
        pltpu.sync_copy(zero_v, acc_sh.at[pl.ds(s * RPT + k * CH, CH)])
    plsc.subcore_barrier()

    def _step(j, _):
        pltpu.sync_copy(dst_hbm.at[wid, j], idx_v)
        pltpu.sync_copy(ones_v, acc_sh.at[idx_v], add=True)
        return 0

    lax.fori_loop(0, NCHUNK, _step, 0)
    plsc.subcore_barrier()
    pltpu.sync_copy(acc_sh.at[pl.ds(s * RPT, RPT)],
                    out_hbm.at[c, pl.ds(s * RPT, RPT)])


_deg_call = pl.kernel(
    _deg_body,
    out_type=jax.ShapeDtypeStruct((NC, NPAD, 16), jnp.float32),
    mesh=_mesh,
    scratch_types=[
        pltpu.VMEM((CH,), jnp.int32),
        pltpu.VMEM((CH, 16), jnp.float32),
        pltpu.VMEM((CH, 16), jnp.float32),
        pltpu.VMEM_SHARED((NPAD, 16), jnp.float32),
        pltpu.SemaphoreType.DMA,
    ],
)


def _agg_body(h_hbm, src_hbm, dst_hbm, out_hbm, sidx_v, didx_v, rows_v,
              acc_sh, sem):
    c = lax.axis_index("c")
    s = lax.axis_index("s")
    wid = c * NS + s

    def _zrow(i, _):
        def _z16(k, _):
            rows_v[i, pl.ds(k * 16, 16)] = jnp.zeros((16,), jnp.float32)
            return 0
        lax.fori_loop(0, D // 16, _z16, 0)
        return 0

    lax.fori_loop(0, CH, _zrow, 0)
    for k in range(ZC):
        pltpu.sync_copy(rows_v, acc_sh.at[pl.ds(s * RPT + k * CH, CH)])
    plsc.subcore_barrier()

    def _step(j, _):
        pltpu.sync_copy(src_hbm.at[wid, j], sidx_v)
        pltpu.sync_copy(dst_hbm.at[wid, j], didx_v)
        pltpu.async_copy(h_hbm.at[sidx_v], rows_v, sem).wait()
        pltpu.sync_copy(rows_v, acc_sh.at[didx_v], add=True)
        return 0

    lax.fori_loop(0, NCHUNK, _step, 0)
    plsc.subcore_barrier()
    pltpu.sync_copy(acc_sh.at[pl.ds(s * RPT, RPT)],
                    out_hbm.at[c, pl.ds(s * RPT, RPT)])


_agg_call = pl.kernel(
    _agg_body,
    out_type=jax.ShapeDtypeStruct((NC, NPAD, D), jnp.float32),
    mesh=_mesh,
    scratch_types=[
        pltpu.VMEM((CH,), jnp.int32),
        pltpu.VMEM((CH,), jnp.int32),
        pltpu.VMEM((CH, D), jnp.float32),
        pltpu.VMEM_SHARED((NPAD, D), jnp.float32),
        pltpu.SemaphoreType.DMA,
    ],
)


def _dis_from(deg_blk):
    deg = deg_blk[0, :, 0:1] + deg_blk[1, :, 0:1] + 1.0
    return lax.rsqrt(deg)


def _tc1_body(x_ref, w_ref, deg_ref, o_ref):
    dis = _dis_from(deg_ref[...])
    o_ref[...] = jnp.dot(x_ref[...], w_ref[...],
                         preferred_element_type=jnp.float32) * dis


def _tc2_body(acc_ref, hp_ref, deg_ref, b_ref, w_ref, o_ref):
    dis = _dis_from(deg_ref[...])
    agg = acc_ref[0] + acc_ref[1] + hp_ref[...]
    out1 = jnp.maximum(dis * agg + b_ref[...], 0.0)
    o_ref[...] = jnp.dot(out1, w_ref[...],
                         preferred_element_type=jnp.float32) * dis


def _tc3_body(acc_ref, hp_ref, deg_ref, b_ref, o_ref):
    dis = _dis_from(deg_ref[...])
    o_ref[...] = dis * (acc_ref[0] + acc_ref[1] + hp_ref[...]) + b_ref[...]


_deg_spec = pl.BlockSpec((2, RB, 16), lambda i: (0, i, 0))
_acc_spec = pl.BlockSpec((2, RB, D), lambda i: (0, i, 0))
_row_spec = pl.BlockSpec((RB, D), lambda i: (i, 0))
_mat_spec = pl.BlockSpec((D, D), lambda i: (0, 0))
_vec_spec = pl.BlockSpec((1, D), lambda i: (0, 0))
_f32 = functools.partial(jax.ShapeDtypeStruct, dtype=jnp.float32)

_tc1_call = pl.pallas_call(
    _tc1_body, grid=(GRID,),
    in_specs=[_row_spec, _mat_spec, _deg_spec],
    out_specs=_row_spec, out_shape=_f32(shape=(N, D)))

_tc2_call = pl.pallas_call(
    _tc2_body, grid=(GRID,),
    in_specs=[_acc_spec, _row_spec, _deg_spec, _vec_spec, _mat_spec],
    out_specs=_row_spec, out_shape=_f32(shape=(N, D)))

_tc3_call = pl.pallas_call(
    _tc3_body, grid=(GRID,),
    in_specs=[_acc_spec, _row_spec, _deg_spec, _vec_spec],
    out_specs=_row_spec, out_shape=_f32(shape=(N, D)))


@jax.jit
def _run(x, src, dst, W1, b1, W2, b2):
    degacc = _deg_call(dst)
    hp1 = _tc1_call(x, W1, degacc)
    acc1 = _agg_call(hp1, src, dst)
    hp2 = _tc2_call(acc1, hp1, degacc, b1.reshape(1, D), W2)
    acc2 = _agg_call(hp2, src, dst)
    return _tc3_call(acc2, hp2, degacc, b2.reshape(1, D))


def kernel(x, edge_index, W1, b1, W2, b2):
    src = edge_index[0].astype(jnp.int32)
    dst = edge_index[1].astype(jnp.int32)
    epad = NW * NCHUNK * CH
    src = jnp.concatenate([src, jnp.zeros((epad - E,), jnp.int32)])
    dst = jnp.concatenate([dst, jnp.full((epad - E,), N, jnp.int32)])
    src = src.reshape(NW, NCHUNK, CH)
    dst = dst.reshape(NW, NCHUNK, CH)
    return _run(x, src, dst, W1, b1, W2, b2)


# speedup vs baseline: 10.5214x; 10.5214x over previous
import functools

import jax
import jax.numpy as jnp
from jax import lax
from jax.experimental import pallas as pl
from jax.experimental.pallas import tpu as pltpu
from jax.experimental.pallas import tpu_sc as plsc

N = 10000
E = 320000
D = 128
NC = 2
NS = 16
NW = NC * NS
CH = 128
EPW = E // NW
NCHUNK = -(-EPW // CH)
NPAD = 10240
RPT = NPAD // NS
ZC = RPT // CH
RB = 512
GRID = NPAD // RB

_mesh = plsc.VectorSubcoreMesh(core_axis_name="c", subcore_axis_name="s",
                               num_cores=NC, num_subcores=NS)


def _deg_body(dst_hbm, out_hbm, idx_v, acc_v, sem):
    c = lax.axis_index("c")
    s = lax.axis_index("s")
    wid = c * NS + s
    ones = jnp.ones((16,), jnp.float32)

    def _z(i, _):
        acc_v[pl.ds(i * 16, 16)] = jnp.zeros((16,), jnp.float32)
        return 0

    lax.fori_loop(0, NPAD // 16, _z, 0)

    def _step(j, _):
        pltpu.sync_copy(dst_hbm.at[wid, j], idx_v)

        def _grp(k, _):
            idx = idx_v[pl.ds(k * 16, 16)]
            plsc.addupdate_scatter(acc_v, [idx], ones)
            return 0

        lax.fori_loop(0, CH // 16, _grp, 0)
        return 0

    lax.fori_loop(0, NCHUNK, _step, 0)
    pltpu.sync_copy(acc_v, out_hbm.at[wid])


_deg_call = pl.kernel(
    _deg_body,
    out_type=jax.ShapeDtypeStruct((NW, NPAD), jnp.float32),
    mesh=_mesh,
    scratch_types=[
        pltpu.VMEM((CH,), jnp.int32),
        pltpu.VMEM((NPAD,), jnp.float32),
        pltpu.SemaphoreType.DMA,
    ],
    compiler_params=pltpu.CompilerParams(needs_layout_passes=False),
)


def _agg_body(h_hbm, src_hbm, dst_hbm, out_hbm, sidx_v, didx_v, rows_v,
              acc_sh, sem):
    c = lax.axis_index("c")
    s = lax.axis_index("s")
    wid = c * NS + s

    def _zrow(i, _):
        def _z16(k, _):
            rows_v[i, pl.ds(k * 16, 16)] = jnp.zeros((16,), jnp.float32)
            return 0
        lax.fori_loop(0, D // 16, _z16, 0)
        return 0

    lax.fori_loop(0, CH, _zrow, 0)
    for k in range(ZC):
        pltpu.sync_copy(rows_v, acc_sh.at[pl.ds(s * RPT + k * CH, CH)])
    plsc.subcore_barrier()

    def _step(j, _):
        pltpu.sync_copy(src_hbm.at[wid, j], sidx_v)
        pltpu.sync_copy(dst_hbm.at[wid, j], didx_v)
        pltpu.async_copy(h_hbm.at[sidx_v], rows_v, sem).wait()
        pltpu.sync_copy(rows_v, acc_sh.at[didx_v], add=True)
        return 0

    lax.fori_loop(0, NCHUNK, _step, 0)
    plsc.subcore_barrier()
    pltpu.sync_copy(acc_sh.at[pl.ds(s * RPT, RPT)],
                    out_hbm.at[c, pl.ds(s * RPT, RPT)])


_agg_call = pl.kernel(
    _agg_body,
    out_type=jax.ShapeDtypeStruct((NC, NPAD, D), jnp.float32),
    mesh=_mesh,
    scratch_types=[
        pltpu.VMEM((CH,), jnp.int32),
        pltpu.VMEM((CH,), jnp.int32),
        pltpu.VMEM((CH, D), jnp.float32),
        pltpu.VMEM_SHARED((NPAD, D), jnp.float32),
        pltpu.SemaphoreType.DMA,
    ],
)


def _dis_from(deg_blk):
    deg = jnp.sum(deg_blk, axis=0) + 1.0
    return lax.rsqrt(deg).reshape(RB, 1)


def _tc1_body(x_ref, w_ref, deg_ref, o_ref):
    dis = _dis_from(deg_ref[...])
    o_ref[...] = jnp.dot(x_ref[...], w_ref[...],
                         preferred_element_type=jnp.float32) * dis


def _tc2_body(acc_ref, hp_ref, deg_ref, b_ref, w_ref, o_ref):
    dis = _dis_from(deg_ref[...])
    agg = acc_ref[0] + acc_ref[1] + hp_ref[...]
    out1 = jnp.maximum(dis * agg + b_ref[...], 0.0)
    o_ref[...] = jnp.dot(out1, w_ref[...],
                         preferred_element_type=jnp.float32) * dis


def _tc3_body(acc_ref, hp_ref, deg_ref, b_ref, o_ref):
    dis = _dis_from(deg_ref[...])
    o_ref[...] = dis * (acc_ref[0] + acc_ref[1] + hp_ref[...]) + b_ref[...]


_deg_spec = pl.BlockSpec((NW, RB), lambda i: (0, i))
_acc_spec = pl.BlockSpec((2, RB, D), lambda i: (0, i, 0))
_row_spec = pl.BlockSpec((RB, D), lambda i: (i, 0))
_mat_spec = pl.BlockSpec((D, D), lambda i: (0, 0))
_vec_spec = pl.BlockSpec((1, D), lambda i: (0, 0))
_f32 = functools.partial(jax.ShapeDtypeStruct, dtype=jnp.float32)

_tc1_call = pl.pallas_call(
    _tc1_body, grid=(GRID,),
    in_specs=[_row_spec, _mat_spec, _deg_spec],
    out_specs=_row_spec, out_shape=_f32(shape=(NPAD, D)))

_tc2_call = pl.pallas_call(
    _tc2_body, grid=(GRID,),
    in_specs=[_acc_spec, _row_spec, _deg_spec, _vec_spec, _mat_spec],
    out_specs=_row_spec, out_shape=_f32(shape=(NPAD, D)))

_tc3_call = pl.pallas_call(
    _tc3_body, grid=(GRID,),
    in_specs=[_acc_spec, _row_spec, _deg_spec, _vec_spec],
    out_specs=_row_spec, out_shape=_f32(shape=(NPAD, D)))


@jax.jit
def _run(x, src, dst, W1, b1, W2, b2):
    degacc = _deg_call(dst)
    hp1 = _tc1_call(x, W1, degacc)
    acc1 = _agg_call(hp1, src, dst)
    hp2 = _tc2_call(acc1, hp1, degacc, b1.reshape(1, D), W2)
    acc2 = _agg_call(hp2, src, dst)
    return _tc3_call(acc2, hp2, degacc, b2.reshape(1, D))


def kernel(x, edge_index, W1, b1, W2, b2):
    x = jnp.concatenate([x, jnp.zeros((NPAD - N, D), jnp.float32)])
    src = edge_index[0].astype(jnp.int32)
    dst = edge_index[1].astype(jnp.int32)
    epad = NW * NCHUNK * CH
    src = jnp.concatenate([src, jnp.zeros((epad - E,), jnp.int32)])
    dst = jnp.concatenate([dst, jnp.full((epad - E,), N, jnp.int32)])
    src = src.reshape(NW, NCHUNK, CH)
    dst = dst.reshape(NW, NCHUNK, CH)
    return _run(x, src, dst, W1, b1, W2, b2)[:N]
